# baseline (speedup 1.0000x reference)
import functools

import jax
import jax.numpy as jnp
from jax import lax
from jax.experimental import pallas as pl
from jax.experimental.pallas import tpu as pltpu
from jax.experimental.pallas import tpu_sc as plsc

_K = 8
_DVAR = 1.0
_DDIST = 2.0
_NC = 2
_NS = 16
_NW = _NC * _NS
_L = 16


def _make_seg_sum(b, d, h, w):
    rows = h // _NW
    mesh = plsc.VectorSubcoreMesh(core_axis_name="c", subcore_axis_name="s")

    @functools.partial(
        pl.kernel, mesh=mesh,
        compiler_params=pltpu.CompilerParams(needs_layout_passes=False),
        out_type=jax.ShapeDtypeStruct((_NW, _K * d), jnp.float32),
        scratch_types=[
            pltpu.VMEM((rows, w), jnp.int32),
            pltpu.VMEM((2, rows, w), jnp.float32),
            pltpu.VMEM((_K * d,), jnp.float32),
            pltpu.SemaphoreType.DMA,
            pltpu.SemaphoreType.DMA,
            pltpu.SemaphoreType.DMA,
        ],
    )
    def seg_sum(data_hbm, lab_hbm, out_hbm, lab_v, xbuf, acc, lsem, sem0, sem1):
        wid = lax.axis_index("s") * _NC + lax.axis_index("c")
        h0 = wid * rows
        lab_cp = pltpu.async_copy(lab_hbm.at[b - 1, pl.ds(h0, rows), :],
                                  lab_v, lsem)
        for k in range(_K * d // _L):
            acc[pl.ds(k * _L, _L)] = jnp.zeros((_L,), jnp.float32)
        sems = (sem0, sem1)
        pend = pltpu.async_copy(data_hbm.at[b - 1, 0, pl.ds(h0, rows), :],
                                xbuf.at[0], sems[0])
        lab_cp.wait()

        def scale_row(i, _):
            def scale_px(j, _):
                lab_v[i, pl.ds(j * _L, _L)] = (
                    lab_v[i, pl.ds(j * _L, _L)] * d)
                return 0
            lax.fori_loop(0, w // _L, scale_px, 0)
            return 0
        lax.fori_loop(0, rows, scale_row, 0)

        def make_row(dd, buf):
            def row_body(i, _):
                def px_body(j, _):
                    v = xbuf[buf, i, pl.ds(j * _L, _L)]
                    lv = lab_v[i, pl.ds(j * _L, _L)]
                    plsc.addupdate_scatter(acc, [lv + dd], v)
                    return 0
                lax.fori_loop(0, w // _L, px_body, 0)
                return 0
            return row_body

        for dd in range(d):
            pend.wait()
            if dd + 1 < d:
                pend = pltpu.async_copy(
                    data_hbm.at[b - 1, dd + 1, pl.ds(h0, rows), :],
                    xbuf.at[(dd + 1) % 2], sems[(dd + 1) % 2])
            lax.fori_loop(0, rows, make_row(dd, dd % 2), 0)
        pltpu.sync_copy(acc, out_hbm.at[wid])

    return seg_sum


def _tc_body(part_ref, lab_ref, x_ref, out_ref, mu_ref, loss_ref):
    c = pl.program_id(0)
    nc = pl.num_programs(0)
    x = x_ref[0]
    lab = lab_ref[0]
    d, hc, w = x.shape
    n = hc * w * nc
    onehot = (lax.broadcasted_iota(jnp.int32, (_K, hc, w), 0) ==
              lab[None]).astype(jnp.float32).reshape(_K, hc * w)
    x = x.reshape(d, hc * w)

    @pl.when(c == 0)
    def _mid():
        mu = jnp.sum(part_ref[...], axis=0) * (1.0 / n)
        mu_ref[...] = mu
        g = lax.dot_general(mu, mu, (((1,), (1,)), ((), ())),
                            preferred_element_type=jnp.float32)
        eye = (lax.broadcasted_iota(jnp.int32, (_K, _K), 0) ==
               lax.broadcasted_iota(jnp.int32, (_K, _K), 1)).astype(jnp.float32)
        dr = jnp.sum(g * eye, axis=1, keepdims=True)
        dc = jnp.sum(g * eye, axis=0, keepdims=True)
        d2 = jnp.maximum(dr + dc - 2.0 * g, 0.0)
        dist = jnp.maximum(_DDIST - jnp.sqrt(d2), 0.0) ** 2
        loss_ref[0, 0] = jnp.sum(dist) / (_K - 1) / 2.0

    musel = lax.dot_general(
        mu_ref[...], onehot, (((0,), (0,)), ((), ())),
        preferred_element_type=jnp.float32)
    diff = x - musel
    r2 = jnp.sum(diff * diff, axis=0, keepdims=True)
    t = jnp.maximum(jnp.sqrt(r2) - _DVAR, 0.0) ** 2
    loss_ref[0, 0] += jnp.sum(t) / n

    @pl.when(c == nc - 1)
    def _fin():
        out_ref[...] = jnp.full((1, 1), loss_ref[0, 0], jnp.float32)


def kernel(data, labels):
    b, d, h, w = data.shape
    partials = _make_seg_sum(b, d, h, w)(data, labels).reshape(_NW, _K, d)
    nchunks = 8
    hc = h // nchunks
    out = pl.pallas_call(
        _tc_body,
        grid=(nchunks,),
        in_specs=[
            pl.BlockSpec((_NW, _K, d), lambda c: (0, 0, 0)),
            pl.BlockSpec((1, hc, w), lambda c: (b - 1, c, 0)),
            pl.BlockSpec((1, d, hc, w), lambda c: (b - 1, 0, c, 0)),
        ],
        out_specs=pl.BlockSpec((1, 1), lambda c: (0, 0)),
        out_shape=jax.ShapeDtypeStruct((1, 1), jnp.float32),
        scratch_shapes=[
            pltpu.VMEM((_K, d), jnp.float32),
            pltpu.SMEM((1, 1), jnp.float32),
        ],
        compiler_params=pltpu.CompilerParams(
            dimension_semantics=("arbitrary",)),
    )(partials, labels, data)
    return out[0, 0]


# speedup vs baseline: 1.9045x; 1.9045x over previous
import functools

import jax
import jax.numpy as jnp
from jax import lax
from jax.experimental import pallas as pl
from jax.experimental.pallas import tpu as pltpu
from jax.experimental.pallas import tpu_sc as plsc

_K = 8
_DVAR = 1.0
_DDIST = 2.0
_NC = 2
_NS = 16
_NW = _NC * _NS
_L = 16


_BS = 129


def _make_seg_sum(b, d, h, w):
    rows = h // _NW
    qrows = 4
    nq = rows // qrows
    mesh = plsc.VectorSubcoreMesh(core_axis_name="c", subcore_axis_name="s")

    @functools.partial(
        pl.kernel, mesh=mesh,
        compiler_params=pltpu.CompilerParams(needs_layout_passes=False),
        out_type=jax.ShapeDtypeStruct((_NW, _K * d), jnp.float32),
        scratch_types=[
            pltpu.VMEM((rows, w), jnp.int32),
            pltpu.VMEM((2, d, qrows, w), jnp.float32),
            pltpu.VMEM((_L * _BS,), jnp.float32),
            pltpu.VMEM((_K * d,), jnp.float32),
            pltpu.SemaphoreType.DMA,
            pltpu.SemaphoreType.DMA,
            pltpu.SemaphoreType.DMA,
        ],
    )
    def seg_sum(data_hbm, lab_hbm, out_hbm, lab_v, xbuf, acc, accsum,
                lsem, sem0, sem1):
        wid = lax.axis_index("s") * _NC + lax.axis_index("c")
        h0 = wid * rows
        lab_cp = pltpu.async_copy(lab_hbm.at[b - 1, pl.ds(h0, rows), :],
                                  lab_v, lsem)
        for t in range(_L * _BS // _L):
            acc[pl.ds(t * _L, _L)] = jnp.zeros((_L,), jnp.float32)
        sems = (sem0, sem1)
        pend = pltpu.async_copy(data_hbm.at[b - 1, :, pl.ds(h0, qrows), :],
                                xbuf.at[0], sems[0])
        lab_cp.wait()
        bank = lax.iota(jnp.int32, _L) * _BS

        for q in range(nq):
            pend.wait()
            if q + 1 < nq:
                pend = pltpu.async_copy(
                    data_hbm.at[b - 1, :, pl.ds(h0 + (q + 1) * qrows, qrows), :],
                    xbuf.at[(q + 1) % 2], sems[(q + 1) % 2])
            buf = q % 2

            def row_body(i, _, q=q, buf=buf):
                def px_body(j, _):
                    lv = lab_v[q * qrows + i, pl.ds(j * _L, _L)]
                    idx = lv * d + bank
                    for dd in range(d):
                        v = xbuf[buf, dd, i, pl.ds(j * _L, _L)]
                        plsc.addupdate_scatter(acc, [idx + dd], v)
                    return 0
                lax.fori_loop(0, w // _L, px_body, 0)
                return 0
            lax.fori_loop(0, qrows, row_body, 0)

        for r in range(_K * d // _L):
            s = acc[pl.ds(r * _L, _L)]
            for k in range(1, _L):
                s = s + acc[pl.ds(k * _BS + r * _L, _L)]
            accsum[pl.ds(r * _L, _L)] = s
        pltpu.sync_copy(accsum, out_hbm.at[wid])

    return seg_sum


def _tc_body(part_ref, lab_ref, x_ref, out_ref, mu_ref, loss_ref):
    c = pl.program_id(0)
    nc = pl.num_programs(0)
    x = x_ref[0]
    lab = lab_ref[0]
    d, hc, w = x.shape
    n = hc * w * nc
    onehot = (lax.broadcasted_iota(jnp.int32, (_K, hc, w), 0) ==
              lab[None]).astype(jnp.float32).reshape(_K, hc * w)
    x = x.reshape(d, hc * w)

    @pl.when(c == 0)
    def _mid():
        mu = jnp.sum(part_ref[...], axis=0) * (1.0 / n)
        mu_ref[...] = mu
        g = lax.dot_general(mu, mu, (((1,), (1,)), ((), ())),
                            preferred_element_type=jnp.float32)
        eye = (lax.broadcasted_iota(jnp.int32, (_K, _K), 0) ==
               lax.broadcasted_iota(jnp.int32, (_K, _K), 1)).astype(jnp.float32)
        dr = jnp.sum(g * eye, axis=1, keepdims=True)
        dc = jnp.sum(g * eye, axis=0, keepdims=True)
        d2 = jnp.maximum(dr + dc - 2.0 * g, 0.0)
        dist = jnp.maximum(_DDIST - jnp.sqrt(d2), 0.0) ** 2
        loss_ref[0, 0] = jnp.sum(dist) / (_K - 1) / 2.0

    musel = lax.dot_general(
        mu_ref[...], onehot, (((0,), (0,)), ((), ())),
        preferred_element_type=jnp.float32)
    diff = x - musel
    r2 = jnp.sum(diff * diff, axis=0, keepdims=True)
    t = jnp.maximum(jnp.sqrt(r2) - _DVAR, 0.0) ** 2
    loss_ref[0, 0] += jnp.sum(t) / n

    @pl.when(c == nc - 1)
    def _fin():
        out_ref[...] = jnp.full((1, 1), loss_ref[0, 0], jnp.float32)


def kernel(data, labels):
    b, d, h, w = data.shape
    partials = _make_seg_sum(b, d, h, w)(data, labels).reshape(_NW, _K, d)
    nchunks = 8
    hc = h // nchunks
    out = pl.pallas_call(
        _tc_body,
        grid=(nchunks,),
        in_specs=[
            pl.BlockSpec((_NW, _K, d), lambda c: (0, 0, 0)),
            pl.BlockSpec((1, hc, w), lambda c: (b - 1, c, 0)),
            pl.BlockSpec((1, d, hc, w), lambda c: (b - 1, 0, c, 0)),
        ],
        out_specs=pl.BlockSpec((1, 1), lambda c: (0, 0)),
        out_shape=jax.ShapeDtypeStruct((1, 1), jnp.float32),
        scratch_shapes=[
            pltpu.VMEM((_K, d), jnp.float32),
            pltpu.SMEM((1, 1), jnp.float32),
        ],
        compiler_params=pltpu.CompilerParams(
            dimension_semantics=("arbitrary",)),
    )(partials, labels, data)
    return out[0, 0]


# speedup vs baseline: 2.8027x; 1.4717x over previous
import functools

import jax
import jax.numpy as jnp
from jax import lax
from jax.experimental import pallas as pl
from jax.experimental.pallas import tpu as pltpu
from jax.experimental.pallas import tpu_sc as plsc

_K = 8
_DVAR = 1.0
_DDIST = 2.0
_NC = 2
_NS = 16
_NW = _NC * _NS
_L = 16


_BS = 129


def _make_seg_sum(b, d, h, w, hbase, sc_rows):
    rows = sc_rows // _NW
    qrows = 2
    nq = rows // qrows
    mesh = plsc.VectorSubcoreMesh(core_axis_name="c", subcore_axis_name="s")

    @functools.partial(
        pl.kernel, mesh=mesh,
        compiler_params=pltpu.CompilerParams(needs_layout_passes=False),
        out_type=jax.ShapeDtypeStruct((_NW, _K * d), jnp.float32),
        scratch_types=[
            pltpu.VMEM((rows, w), jnp.int32),
            pltpu.VMEM((2, d, qrows, w), jnp.float32),
            pltpu.VMEM((_L * _BS,), jnp.float32),
            pltpu.VMEM((_K * d,), jnp.float32),
            pltpu.SemaphoreType.DMA,
            pltpu.SemaphoreType.DMA,
            pltpu.SemaphoreType.DMA,
        ],
    )
    def seg_sum(data_hbm, lab_hbm, out_hbm, lab_v, xbuf, acc, accsum,
                lsem, sem0, sem1):
        wid = lax.axis_index("s") * _NC + lax.axis_index("c")
        h0 = hbase + wid * rows
        lab_cp = pltpu.async_copy(lab_hbm.at[b - 1, pl.ds(h0, rows), :],
                                  lab_v, lsem)
        for t in range(_L * _BS // _L):
            acc[pl.ds(t * _L, _L)] = jnp.zeros((_L,), jnp.float32)
        sems = (sem0, sem1)
        pend = pltpu.async_copy(data_hbm.at[b - 1, :, pl.ds(h0, qrows), :],
                                xbuf.at[0], sems[0])
        lab_cp.wait()
        bank = lax.iota(jnp.int32, _L) * _BS

        for q in range(nq):
            pend.wait()
            if q + 1 < nq:
                pend = pltpu.async_copy(
                    data_hbm.at[b - 1, :, pl.ds(h0 + (q + 1) * qrows, qrows), :],
                    xbuf.at[(q + 1) % 2], sems[(q + 1) % 2])
            buf = q % 2

            def row_body(i, _, q=q, buf=buf):
                def px_body(j, _):
                    lv = lab_v[q * qrows + i, pl.ds(j * _L, _L)]
                    idx = lv * d + bank
                    for dd in range(d):
                        v = xbuf[buf, dd, i, pl.ds(j * _L, _L)]
                        plsc.addupdate_scatter(acc, [idx + dd], v)
                    return 0
                lax.fori_loop(0, w // _L, px_body, 0)
                return 0
            lax.fori_loop(0, qrows, row_body, 0)

        for r in range(_K * d // _L):
            s = acc[pl.ds(r * _L, _L)]
            for k in range(1, _L):
                s = s + acc[pl.ds(k * _BS + r * _L, _L)]
            accsum[pl.ds(r * _L, _L)] = s
        pltpu.sync_copy(accsum, out_hbm.at[wid])

    return seg_sum


def _tc_sums_body(lab_ref, x_ref, out_ref, acc_ref):
    c = pl.program_id(0)
    nc = pl.num_programs(0)
    x = x_ref[0]
    lab = lab_ref[0]
    d, hc, w = x.shape
    onehot = (lax.broadcasted_iota(jnp.int32, (_K, hc, w), 0) ==
              lab[None]).astype(jnp.float32).reshape(_K, hc * w)
    x2 = x.reshape(d, hc * w)

    @pl.when(c == 0)
    def _init():
        acc_ref[...] = jnp.zeros_like(acc_ref)

    acc_ref[...] += lax.dot_general(
        onehot, x2, (((1,), (1,)), ((), ())),
        preferred_element_type=jnp.float32)

    @pl.when(c == nc - 1)
    def _fin():
        out_ref[...] = acc_ref[...]


def _tc_body(part_ref, tcs_ref, lab_ref, x_ref, out_ref, mu_ref, loss_ref):
    c = pl.program_id(0)
    nc = pl.num_programs(0)
    x = x_ref[0]
    lab = lab_ref[0]
    d, hc, w = x.shape
    n = hc * w * nc
    onehot = (lax.broadcasted_iota(jnp.int32, (_K, hc, w), 0) ==
              lab[None]).astype(jnp.float32).reshape(_K, hc * w)
    x = x.reshape(d, hc * w)

    @pl.when(c == 0)
    def _mid():
        mu = ((jnp.sum(part_ref[...], axis=0) + tcs_ref[...])
              * (1.0 / n))
        mu_ref[...] = mu
        g = lax.dot_general(mu, mu, (((1,), (1,)), ((), ())),
                            preferred_element_type=jnp.float32)
        eye = (lax.broadcasted_iota(jnp.int32, (_K, _K), 0) ==
               lax.broadcasted_iota(jnp.int32, (_K, _K), 1)).astype(jnp.float32)
        dr = jnp.sum(g * eye, axis=1, keepdims=True)
        dc = jnp.sum(g * eye, axis=0, keepdims=True)
        d2 = jnp.maximum(dr + dc - 2.0 * g, 0.0)
        dist = jnp.maximum(_DDIST - jnp.sqrt(d2), 0.0) ** 2
        loss_ref[0, 0] = jnp.sum(dist) / (_K - 1) / 2.0

    musel = lax.dot_general(
        mu_ref[...], onehot, (((0,), (0,)), ((), ())),
        preferred_element_type=jnp.float32)
    diff = x - musel
    r2 = jnp.sum(diff * diff, axis=0, keepdims=True)
    t = jnp.maximum(jnp.sqrt(r2) - _DVAR, 0.0) ** 2
    loss_ref[0, 0] += jnp.sum(t) / n

    @pl.when(c == nc - 1)
    def _fin():
        out_ref[...] = jnp.full((1, 1), loss_ref[0, 0], jnp.float32)


def kernel(data, labels):
    b, d, h, w = data.shape
    sc_rows = 128
    tc_rows = h - sc_rows
    partials = _make_seg_sum(b, d, h, w, tc_rows, sc_rows)(data, labels)
    partials = partials.reshape(_NW, _K, d)
    tchunks = 6
    thc = tc_rows // tchunks
    tc_sums = pl.pallas_call(
        _tc_sums_body,
        grid=(tchunks,),
        in_specs=[
            pl.BlockSpec((1, thc, w), lambda c: (b - 1, c, 0)),
            pl.BlockSpec((1, d, thc, w), lambda c: (b - 1, 0, c, 0)),
        ],
        out_specs=pl.BlockSpec((_K, d), lambda c: (0, 0)),
        out_shape=jax.ShapeDtypeStruct((_K, d), jnp.float32),
        scratch_shapes=[pltpu.VMEM((_K, d), jnp.float32)],
        compiler_params=pltpu.CompilerParams(
            dimension_semantics=("arbitrary",)),
    )(labels, data)
    nchunks = 8
    hc = h // nchunks
    out = pl.pallas_call(
        _tc_body,
        grid=(nchunks,),
        in_specs=[
            pl.BlockSpec((_NW, _K, d), lambda c: (0, 0, 0)),
            pl.BlockSpec((_K, d), lambda c: (0, 0)),
            pl.BlockSpec((1, hc, w), lambda c: (b - 1, c, 0)),
            pl.BlockSpec((1, d, hc, w), lambda c: (b - 1, 0, c, 0)),
        ],
        out_specs=pl.BlockSpec((1, 1), lambda c: (0, 0)),
        out_shape=jax.ShapeDtypeStruct((1, 1), jnp.float32),
        scratch_shapes=[
            pltpu.VMEM((_K, d), jnp.float32),
            pltpu.SMEM((1, 1), jnp.float32),
        ],
        compiler_params=pltpu.CompilerParams(
            dimension_semantics=("arbitrary",)),
    )(partials, tc_sums, labels, data)
    return out[0, 0]
